# Initial kernel scaffold; baseline (speedup 1.0000x reference)
import jax
import jax.numpy as jnp
from jax.experimental import pallas as pl


def kernel(points_2d, camera_indices, point_indices, camera_params, points_3d):
    raise NotImplementedError("write your pallas kernel here")


# speedup vs baseline: 2.6882x; 2.6882x over previous
import jax
import jax.numpy as jnp
from jax import lax
from jax.experimental import pallas as pl
from jax.experimental.pallas import tpu as pltpu
from jax.experimental.pallas import tpu_sc as plsc

N_OBS = 800_000
N_CAM = 1000
N_PTS = 200_000
NW = 32
CHUNK = 1280
N_CHUNKS = N_OBS // CHUNK
ITERS = -(-N_CHUNKS // NW)
STEPS = CHUNK // 16
IDX_SUB = 128
N_SUB = CHUNK // IDX_SUB
ROWS_A = 12504
ROWS_LAST = N_PTS - 15 * ROWS_A


def _body(p2d_hbm, cidx_hbm, pidx_hbm, cam_hbm, pts_hbm, out_hbm,
          pts_sh, cam_tab, cidx_v, pidx_v, p2d_v, pts_v, out_v, sem):
    c = lax.axis_index("c")
    s = lax.axis_index("s")
    wid = s * 2 + c

    pltpu.sync_copy(cam_hbm, cam_tab)

    @pl.when(s < 15)
    def _():
        pltpu.sync_copy(pts_hbm.at[pl.ds(s * ROWS_A, ROWS_A)],
                        pts_sh.at[pl.ds(s * ROWS_A, ROWS_A)])

    @pl.when(s == 15)
    def _():
        pltpu.sync_copy(pts_hbm.at[pl.ds(15 * ROWS_A, ROWS_LAST)],
                        pts_sh.at[pl.ds(15 * ROWS_A, ROWS_LAST)])

    plsc.subcore_barrier()

    lane = lax.iota(jnp.int32, 16)
    col = [jnp.full((16,), j, jnp.int32) for j in range(10)]

    def chunk_body(k, carry):
        cid = wid + k * NW

        @pl.when(cid < N_CHUNKS)
        def _():
            base = cid * CHUNK
            pltpu.sync_copy(cidx_hbm.at[pl.ds(base, CHUNK)], cidx_v)
            pltpu.sync_copy(pidx_hbm.at[cid], pidx_v)
            pltpu.sync_copy(p2d_hbm.at[pl.ds(base, CHUNK)], p2d_v)
            copies = [
                pltpu.async_copy(pts_sh.at[pidx_v.at[j]],
                                 pts_v.at[pl.ds(j * IDX_SUB, IDX_SUB)], sem)
                for j in range(N_SUB)
            ]
            for cp in copies:
                cp.wait()

            def step(i, carry2):
                ridx = i * 16 + lane
                ci = cidx_v[pl.ds(i * 16, 16)]
                qw = plsc.load_gather(cam_tab, [ci, col[0]])
                qx = plsc.load_gather(cam_tab, [ci, col[1]])
                qy = plsc.load_gather(cam_tab, [ci, col[2]])
                qz = plsc.load_gather(cam_tab, [ci, col[3]])
                trx = plsc.load_gather(cam_tab, [ci, col[4]])
                try_ = plsc.load_gather(cam_tab, [ci, col[5]])
                trz = plsc.load_gather(cam_tab, [ci, col[6]])
                f = plsc.load_gather(cam_tab, [ci, col[7]])
                k1 = plsc.load_gather(cam_tab, [ci, col[8]])
                k2 = plsc.load_gather(cam_tab, [ci, col[9]])
                px = plsc.load_gather(pts_v, [ridx, col[0]])
                py = plsc.load_gather(pts_v, [ridx, col[1]])
                pz = plsc.load_gather(pts_v, [ridx, col[2]])
                u2 = plsc.load_gather(p2d_v, [ridx, col[0]])
                v2 = plsc.load_gather(p2d_v, [ridx, col[1]])

                n2 = qw * qw + qx * qx + qy * qy + qz * qz
                s2 = 2.0 / n2
                tx = qy * pz - qz * py + qw * px
                ty = qz * px - qx * pz + qw * py
                tz = qx * py - qy * px + qw * pz
                rx = px + s2 * (qy * tz - qz * ty) + trx
                ry = py + s2 * (qz * tx - qx * tz) + try_
                rz = pz + s2 * (qx * ty - qy * tx) + trz
                u = -rx / rz
                v = -ry / rz
                n = u * u + v * v
                r = 1.0 + k1 * n + k2 * n * n
                fr = f * r
                plsc.store_scatter(out_v, [ridx, col[0]], u * fr - u2)
                plsc.store_scatter(out_v, [ridx, col[1]], v * fr - v2)
                return carry2

            lax.fori_loop(0, STEPS, step, 0)
            pltpu.sync_copy(out_v, out_hbm.at[pl.ds(base, CHUNK)])

        return carry

    lax.fori_loop(0, ITERS, chunk_body, 0)


_mesh = plsc.VectorSubcoreMesh(core_axis_name="c", subcore_axis_name="s")

_proj = pl.kernel(
    _body,
    out_type=jax.ShapeDtypeStruct((N_OBS, 2), jnp.float32),
    mesh=_mesh,
    scratch_types=[
        pltpu.VMEM_SHARED((N_PTS, 3), jnp.float32),
        pltpu.VMEM((N_CAM, 10), jnp.float32),
        pltpu.VMEM((CHUNK,), jnp.int32),
        pltpu.VMEM((N_SUB, IDX_SUB), jnp.int32),
        pltpu.VMEM((CHUNK, 2), jnp.float32),
        pltpu.VMEM((CHUNK, 3), jnp.float32),
        pltpu.VMEM((CHUNK, 2), jnp.float32),
        pltpu.SemaphoreType.DMA,
    ],
    compiler_params=pltpu.CompilerParams(
        needs_layout_passes=False, use_tc_tiling_on_sc=False),
)


def kernel(points_2d, camera_indices, point_indices, camera_params, points_3d):
    ci = camera_indices.astype(jnp.int32)
    pi = point_indices.astype(jnp.int32).reshape(N_CHUNKS, N_SUB, IDX_SUB)
    return _proj(points_2d, ci, pi, camera_params, points_3d)
